# baseline (speedup 1.0000x reference)
import functools

import jax
import jax.numpy as jnp
from jax import lax
from jax.experimental import pallas as pl
from jax.experimental.pallas import tpu as pltpu
from jax.experimental.pallas import tpu_sc as plsc

_B = 2
_CQ = 256
_DIM = 16
_V = _DIM * _DIM * _DIM
_HID = 128
_HEADS = 8
_PTS = 4
_SCALES = 3
_HD = 16
_OFFSET_SCALE = 0.35
_EPS = 1e-5
_Q = _B * _HEADS * _V
_NW = 32
_SLOTS_PW = _Q // _NW
_CHUNK = 64
_NCHUNK = _SLOTS_PW // _CHUNK

_SCALE_DIMS = (64, 32, 16)
_SCALE_CH = (64, 128, 256)


def _tanh(x):
    a = jnp.minimum(jnp.abs(x), 20.0)
    t = 1.0 - 2.0 / (jnp.exp(2.0 * a) + 1.0)
    return jnp.sign(x) * t


def _prep_kernel(s_i, qf_ref, wq_ref, woff_ref, boff_ref, ww_ref, bw_ref,
                 idx_ref, wt_ref):
    b = pl.program_id(0)
    gs = _SCALE_DIMS[s_i]
    svox = gs * gs * gs
    qfb = qf_ref[0]

    w1 = jnp.concatenate([woff_ref[...], ww_ref[...]], axis=0)
    query = lax.dot_general(wq_ref[...], qfb, (((1,), (0,)), ((), ())),
                            preferred_element_type=jnp.float32)
    raw = lax.dot_general(w1, query, (((1,), (0,)), ((), ())),
                          preferred_element_type=jnp.float32)
    bias = jnp.concatenate([boff_ref[...], bw_ref[...]], axis=0)
    raw = raw + bias[:, None]

    off3 = raw[: _HEADS * 36].reshape(_HEADS, 36, _V)
    logits = raw[_HEADS * 36:].reshape(_HEADS, 12, _V)
    m = jnp.max(logits, axis=1, keepdims=True)
    e = jnp.exp(logits - m)
    attn = e / jnp.sum(e, axis=1, keepdims=True)

    vi = lax.broadcasted_iota(jnp.int32, (1, _V), 1)
    wi = vi % _DIM
    hi = (vi // _DIM) % _DIM
    di = vi // (_DIM * _DIM)
    step = 2.0 / (_DIM - 1.0)
    xn = wi.astype(jnp.float32) * step - 1.0
    yn = hi.astype(jnp.float32) * step - 1.0
    zn = di.astype(jnp.float32) * step - 1.0
    hvec = lax.broadcasted_iota(jnp.int32, (_HEADS, _V), 0)

    gsf = float(gs)
    idx_js = []
    wt_js = []
    for p in range(_PTS):
        base_c = s_i * 12 + p * 3
        offx = _tanh(off3[:, base_c + 0]) * _OFFSET_SCALE
        offy = _tanh(off3[:, base_c + 1]) * _OFFSET_SCALE
        offz = _tanh(off3[:, base_c + 2]) * _OFFSET_SCALE
        ux = jnp.clip(((xn + offx + 1.0) * gsf - 1.0) * 0.5, 0.0, gsf - 1.0)
        uy = jnp.clip(((yn + offy + 1.0) * gsf - 1.0) * 0.5, 0.0, gsf - 1.0)
        uz = jnp.clip(((zn + offz + 1.0) * gsf - 1.0) * 0.5, 0.0, gsf - 1.0)
        x0f = jnp.floor(ux); fx = ux - x0f
        y0f = jnp.floor(uy); fy = uy - y0f
        z0f = jnp.floor(uz); fz = uz - z0f
        x0 = x0f.astype(jnp.int32); x1 = jnp.minimum(x0 + 1, gs - 1)
        y0 = y0f.astype(jnp.int32); y1 = jnp.minimum(y0 + 1, gs - 1)
        z0 = z0f.astype(jnp.int32); z1 = jnp.minimum(z0 + 1, gs - 1)
        att = attn[:, s_i * _PTS + p]
        for cz in range(2):
            zi = z1 if cz else z0
            tz = fz if cz else 1.0 - fz
            for cy in range(2):
                yi = y1 if cy else y0
                ty = fy if cy else 1.0 - fy
                for cx in range(2):
                    xi = x1 if cx else x0
                    tx = fx if cx else 1.0 - fx
                    lin = (zi * gs + yi) * gs + xi
                    row = (b * svox + lin) * _HEADS + hvec
                    idx_js.append(row)
                    wt_js.append(tz * ty * tx * att)
    nck = _V // _CHUNK
    idx_st = jnp.stack(idx_js, axis=0).reshape(32, _HEADS, nck, _CHUNK)
    wt_st = jnp.stack(wt_js, axis=0).reshape(32, _HEADS, nck, _CHUNK)
    idx_ref[...] = jnp.transpose(idx_st, (1, 2, 0, 3)).reshape(
        _HEADS * nck, 32 * _CHUNK)
    wt_ref[...] = jnp.transpose(wt_st, (1, 2, 0, 3)).reshape(
        _HEADS * nck, 32, _CHUNK)


def _make_prep(s_i):
    grid = (_B,)
    return pl.pallas_call(
        functools.partial(_prep_kernel, s_i),
        grid=grid,
        in_specs=[
            pl.BlockSpec((1, _CQ, _V), lambda b: (b, 0, 0)),
            pl.BlockSpec((_HID, _CQ), lambda b: (0, 0)),
            pl.BlockSpec((_HEADS * 36, _HID), lambda b: (0, 0)),
            pl.BlockSpec((_HEADS * 36,), lambda b: (0,)),
            pl.BlockSpec((_HEADS * 12, _HID), lambda b: (0, 0)),
            pl.BlockSpec((_HEADS * 12,), lambda b: (0,)),
        ],
        out_specs=[
            pl.BlockSpec((_HEADS * _V // _CHUNK, 32 * _CHUNK), lambda b: (b, 0)),
            pl.BlockSpec((_HEADS * _V // _CHUNK, 32, _CHUNK),
                         lambda b: (b, 0, 0)),
        ],
        out_shape=[
            jax.ShapeDtypeStruct((_Q // _CHUNK, 32 * _CHUNK), jnp.int32),
            jax.ShapeDtypeStruct((_Q // _CHUNK, 32, _CHUNK), jnp.float32),
        ],
        compiler_params=pltpu.CompilerParams(
            vmem_limit_bytes=110 * 1024 * 1024),
    )


def _proj_kernel(v_ref, w_ref, o_ref):
    o_ref[0] = lax.dot_general(v_ref[0], w_ref[...], (((0,), (1,)), ((), ())),
                               preferred_element_type=jnp.float32)


def _make_proj(s_i, nblk):
    c = _SCALE_CH[s_i]
    svox = _SCALE_DIMS[s_i] ** 3
    nsteps = svox // nblk
    return pl.pallas_call(
        _proj_kernel,
        grid=(_B, nsteps),
        in_specs=[
            pl.BlockSpec((1, c, nblk), lambda b, i: (b, 0, i)),
            pl.BlockSpec((_HID, c), lambda b, i: (0, 0)),
        ],
        out_specs=pl.BlockSpec((1, nblk, _HID), lambda b, i: (b, i, 0)),
        out_shape=jax.ShapeDtypeStruct((_B, svox, _HID), jnp.float32),
        compiler_params=pltpu.CompilerParams(
            vmem_limit_bytes=110 * 1024 * 1024),
    )


def _sc_body(idx0, wt0, idx1, wt1, idx2, wt2, t0, t1, t2, out_hbm,
             idxb2, wtb2, rows2, outb, sem0, sem1):
    wid = lax.axis_index("c") * 16 + lax.axis_index("s")
    base0 = wid * _SLOTS_PW
    cbase = base0 // _CHUNK

    scales = ((idx0, wt0, t0), (idx1, wt1, t1), (idx2, wt2, t2))
    sems = (sem0, sem1)
    nrows = 32 * _CHUNK

    def fire(cc_f, s_i, p):
        chunkid = jnp.minimum(cbase + cc_f, _Q // _CHUNK - 1)
        idx_h, wt_h, tab = scales[s_i]
        pltpu.sync_copy(idx_h.at[chunkid], idxb2.at[p])
        pltpu.sync_copy(wt_h.at[chunkid], wtb2.at[p])
        pltpu.async_copy(tab.at[idxb2.at[p]], rows2.at[p], sems[p])

    def wait_rows(p):
        pltpu.make_async_copy(t0.at[pl.ds(0, nrows)], rows2.at[p],
                              sems[p]).wait()

    def combine(s_i, p):
        def tb_body(tb, c2):
            t16 = tb * 16
            if s_i == 0:
                accs = tuple(jnp.zeros((16,), jnp.float32) for _ in range(16))
            else:
                accs = tuple(outb[t16 + l] for l in range(16))

            def j_body(j, acc_t):
                wv = wtb2[p, j, pl.ds(t16, 16)]
                rbase = j * _CHUNK + t16
                return tuple(acc_t[l] + rows2[p, rbase + l] * wv[l]
                             for l in range(16))

            accs = lax.fori_loop(0, 32, j_body, accs, unroll=4)
            for l in range(16):
                outb[t16 + l] = accs[l]
            return c2

        lax.fori_loop(0, _CHUNK // 16, tb_body, 0)

    fire(0, 0, 0)

    def pair_body(c2, carry):
        cc = c2 * 2
        for k in range(6):
            cc_k = cc + k // 3
            s_k = k % 3
            p_k = k % 2
            kn = k + 1
            fire(cc + kn // 3, kn % 3, kn % 2)
            wait_rows(p_k)
            combine(s_k, p_k)
            if s_k == 2:
                pltpu.sync_copy(
                    outb, out_hbm.at[pl.ds(base0 + cc_k * _CHUNK, _CHUNK)])
        return carry

    lax.fori_loop(0, _NCHUNK // 2, pair_body, 0)
    wait_rows(0)


@functools.cache
def _make_sc_gather():
    return pl.kernel(
        _sc_body,
        out_type=jax.ShapeDtypeStruct((_Q, _HD), jnp.float32),
        mesh=plsc.VectorSubcoreMesh(core_axis_name="c", subcore_axis_name="s"),
        scratch_types=[
            pltpu.VMEM((2, 32 * _CHUNK), jnp.int32),
            pltpu.VMEM((2, 32, _CHUNK), jnp.float32),
            pltpu.VMEM((2, 32 * _CHUNK, _HD), jnp.float32),
            pltpu.VMEM((_CHUNK, _HD), jnp.float32),
            pltpu.SemaphoreType.DMA,
            pltpu.SemaphoreType.DMA,
        ],
        compiler_params=pltpu.CompilerParams(use_tc_tiling_on_sc=False),
    )


def _out_kernel(samp_ref, wout_ref, gamma_ref, beta_ref, o_ref):
    sb = samp_ref[...].reshape(_HEADS, _V, _HD)
    acc = jnp.zeros((_CQ, _V), jnp.float32)
    for h in range(_HEADS):
        acc = acc + lax.dot_general(
            wout_ref[:, h * _HD:(h + 1) * _HD], sb[h],
            (((1,), (1,)), ((), ())), preferred_element_type=jnp.float32)
    mean = jnp.mean(acc, axis=1, keepdims=True)
    ctr = acc - mean
    var = jnp.mean(ctr * ctr, axis=1, keepdims=True)
    o_ref[0] = ctr / jnp.sqrt(var + _EPS) * gamma_ref[...] + beta_ref[...]


_out_call = pl.pallas_call(
    _out_kernel,
    grid=(_B,),
    in_specs=[
        pl.BlockSpec((_HEADS * _V, _HD), lambda b: (b, 0)),
        pl.BlockSpec((_CQ, _HID), lambda b: (0, 0)),
        pl.BlockSpec((_CQ, 1), lambda b: (0, 0)),
        pl.BlockSpec((_CQ, 1), lambda b: (0, 0)),
    ],
    out_specs=pl.BlockSpec((1, _CQ, _V), lambda b: (b, 0, 0)),
    out_shape=jax.ShapeDtypeStruct((_B, _CQ, _V), jnp.float32),
    compiler_params=pltpu.CompilerParams(vmem_limit_bytes=110 * 1024 * 1024),
)


def kernel(query_feature, value_0, value_1, value_2, Wq, Wv0, Wv1, Wv2,
           Woff, boff, Ww, bw, Wout, gamma, beta):
    qf = query_feature.reshape(_B, _CQ, _V)
    idxs = []
    wts = []
    for s_i in range(_SCALES):
        i_s, w_s = _make_prep(s_i)(qf, Wq, Woff, boff, Ww, bw)
        idxs.append(i_s)
        wts.append(w_s)

    values = (value_0, value_1, value_2)
    wvs = (Wv0, Wv1, Wv2)
    tabs = []
    for s_i in range(_SCALES):
        svox = _SCALE_DIMS[s_i] ** 3
        nblk = min(svox, 8192)
        vflat = values[s_i].reshape(_B, _SCALE_CH[s_i], svox)
        proj = _make_proj(s_i, nblk)(vflat, wvs[s_i])
        tabs.append(proj.reshape(_B * svox * _HEADS, _HD))

    sampled = _make_sc_gather()(idxs[0], wts[0], idxs[1], wts[1],
                                idxs[2], wts[2], tabs[0], tabs[1], tabs[2])

    out = _out_call(sampled, Wout,
                    gamma.reshape(_CQ, 1), beta.reshape(_CQ, 1))
    return out.reshape(_B, _CQ, _DIM, _DIM, _DIM)


# speedup vs baseline: 1.1042x; 1.1042x over previous
import functools

import jax
import jax.numpy as jnp
from jax import lax
from jax.experimental import pallas as pl
from jax.experimental.pallas import tpu as pltpu
from jax.experimental.pallas import tpu_sc as plsc

_B = 2
_CQ = 256
_DIM = 16
_V = _DIM * _DIM * _DIM
_HID = 128
_HEADS = 8
_PTS = 4
_SCALES = 3
_HD = 16
_OFFSET_SCALE = 0.35
_EPS = 1e-5
_Q = _B * _HEADS * _V
_NW = 32
_SLOTS_PW = _Q // _NW
_CHUNK = 64
_NCHUNK = _SLOTS_PW // _CHUNK

_SCALE_DIMS = (64, 32, 16)
_SCALE_CH = (64, 128, 256)


def _tanh(x):
    a = jnp.minimum(jnp.abs(x), 20.0)
    t = 1.0 - 2.0 / (jnp.exp(2.0 * a) + 1.0)
    return jnp.sign(x) * t


def _prep_kernel(s_i, qf_ref, wq_ref, woff_ref, boff_ref, ww_ref, bw_ref,
                 idx_ref, wt_ref):
    b = pl.program_id(0)
    gs = _SCALE_DIMS[s_i]
    svox = gs * gs * gs
    qfb = qf_ref[0]

    w1 = jnp.concatenate([woff_ref[...], ww_ref[...]], axis=0)
    query = lax.dot_general(wq_ref[...], qfb, (((1,), (0,)), ((), ())),
                            preferred_element_type=jnp.float32)
    raw = lax.dot_general(w1, query, (((1,), (0,)), ((), ())),
                          preferred_element_type=jnp.float32)
    bias = jnp.concatenate([boff_ref[...], bw_ref[...]], axis=0)
    raw = raw + bias[:, None]

    off3 = raw[: _HEADS * 36].reshape(_HEADS, 36, _V)
    logits = raw[_HEADS * 36:].reshape(_HEADS, 12, _V)
    m = jnp.max(logits, axis=1, keepdims=True)
    e = jnp.exp(logits - m)
    attn = e / jnp.sum(e, axis=1, keepdims=True)

    vi = lax.broadcasted_iota(jnp.int32, (1, _V), 1)
    wi = vi % _DIM
    hi = (vi // _DIM) % _DIM
    di = vi // (_DIM * _DIM)
    step = 2.0 / (_DIM - 1.0)
    xn = wi.astype(jnp.float32) * step - 1.0
    yn = hi.astype(jnp.float32) * step - 1.0
    zn = di.astype(jnp.float32) * step - 1.0
    hvec = lax.broadcasted_iota(jnp.int32, (_HEADS, _V), 0)

    gsf = float(gs)
    idx_js = []
    wt_js = []
    for p in range(_PTS):
        base_c = s_i * 12 + p * 3
        offx = _tanh(off3[:, base_c + 0]) * _OFFSET_SCALE
        offy = _tanh(off3[:, base_c + 1]) * _OFFSET_SCALE
        offz = _tanh(off3[:, base_c + 2]) * _OFFSET_SCALE
        ux = jnp.clip(((xn + offx + 1.0) * gsf - 1.0) * 0.5, 0.0, gsf - 1.0)
        uy = jnp.clip(((yn + offy + 1.0) * gsf - 1.0) * 0.5, 0.0, gsf - 1.0)
        uz = jnp.clip(((zn + offz + 1.0) * gsf - 1.0) * 0.5, 0.0, gsf - 1.0)
        x0f = jnp.floor(ux); fx = ux - x0f
        y0f = jnp.floor(uy); fy = uy - y0f
        z0f = jnp.floor(uz); fz = uz - z0f
        x0 = x0f.astype(jnp.int32); x1 = jnp.minimum(x0 + 1, gs - 1)
        y0 = y0f.astype(jnp.int32); y1 = jnp.minimum(y0 + 1, gs - 1)
        z0 = z0f.astype(jnp.int32); z1 = jnp.minimum(z0 + 1, gs - 1)
        att = attn[:, s_i * _PTS + p]
        for cz in range(2):
            zi = z1 if cz else z0
            tz = fz if cz else 1.0 - fz
            for cy in range(2):
                yi = y1 if cy else y0
                ty = fy if cy else 1.0 - fy
                for cx in range(2):
                    xi = x1 if cx else x0
                    tx = fx if cx else 1.0 - fx
                    lin = (zi * gs + yi) * gs + xi
                    row = (b * svox + lin) * _HEADS + hvec
                    idx_js.append(row)
                    wt_js.append(tz * ty * tx * att)
    nck = _V // _CHUNK
    idx_st = jnp.stack(idx_js, axis=0).reshape(32, _HEADS, nck, _CHUNK)
    wt_st = jnp.stack(wt_js, axis=0).reshape(32, _HEADS, nck, _CHUNK)
    idx_ref[...] = jnp.transpose(idx_st, (1, 2, 0, 3)).reshape(
        _HEADS * nck, 32 * _CHUNK)
    wt_ref[...] = jnp.transpose(wt_st, (1, 2, 0, 3)).reshape(
        _HEADS * nck, 32, _CHUNK)


def _make_prep(s_i):
    grid = (_B,)
    return pl.pallas_call(
        functools.partial(_prep_kernel, s_i),
        grid=grid,
        in_specs=[
            pl.BlockSpec((1, _CQ, _V), lambda b: (b, 0, 0)),
            pl.BlockSpec((_HID, _CQ), lambda b: (0, 0)),
            pl.BlockSpec((_HEADS * 36, _HID), lambda b: (0, 0)),
            pl.BlockSpec((_HEADS * 36,), lambda b: (0,)),
            pl.BlockSpec((_HEADS * 12, _HID), lambda b: (0, 0)),
            pl.BlockSpec((_HEADS * 12,), lambda b: (0,)),
        ],
        out_specs=[
            pl.BlockSpec((_HEADS * _V // _CHUNK, 32 * _CHUNK), lambda b: (b, 0)),
            pl.BlockSpec((_HEADS * _V // _CHUNK, 32, _CHUNK),
                         lambda b: (b, 0, 0)),
        ],
        out_shape=[
            jax.ShapeDtypeStruct((_Q // _CHUNK, 32 * _CHUNK), jnp.int32),
            jax.ShapeDtypeStruct((_Q // _CHUNK, 32, _CHUNK), jnp.float32),
        ],
        compiler_params=pltpu.CompilerParams(
            vmem_limit_bytes=110 * 1024 * 1024),
    )


def _proj_kernel(v_ref, w_ref, o_ref):
    o_ref[0] = lax.dot_general(v_ref[0], w_ref[...], (((0,), (1,)), ((), ())),
                               preferred_element_type=jnp.float32)


def _make_proj(s_i, nblk):
    c = _SCALE_CH[s_i]
    svox = _SCALE_DIMS[s_i] ** 3
    nsteps = svox // nblk
    return pl.pallas_call(
        _proj_kernel,
        grid=(_B, nsteps),
        in_specs=[
            pl.BlockSpec((1, c, nblk), lambda b, i: (b, 0, i)),
            pl.BlockSpec((_HID, c), lambda b, i: (0, 0)),
        ],
        out_specs=pl.BlockSpec((1, nblk, _HID), lambda b, i: (b, i, 0)),
        out_shape=jax.ShapeDtypeStruct((_B, svox, _HID), jnp.float32),
        compiler_params=pltpu.CompilerParams(
            vmem_limit_bytes=110 * 1024 * 1024),
    )


def _sc_body(idx0, wt0, idx1, wt1, idx2, wt2, t0, t1, t2, out_hbm,
             idxb2, wtb2, rows2, outb, sem0, sem1):
    wid = lax.axis_index("c") * 16 + lax.axis_index("s")
    base0 = wid * _SLOTS_PW
    cbase = base0 // _CHUNK

    scales = ((idx0, wt0, t0), (idx1, wt1, t1), (idx2, wt2, t2))
    sems = (sem0, sem1)
    nrows = 32 * _CHUNK

    def fire(cc_f, s_i, p):
        chunkid = jnp.minimum(cbase + cc_f, _Q // _CHUNK - 1)
        idx_h, wt_h, tab = scales[s_i]
        pltpu.sync_copy(idx_h.at[chunkid], idxb2.at[p])
        pltpu.sync_copy(wt_h.at[chunkid], wtb2.at[p])
        pltpu.async_copy(tab.at[idxb2.at[p]], rows2.at[p], sems[p])

    def wait_rows(p):
        pltpu.make_async_copy(t0.at[pl.ds(0, nrows)], rows2.at[p],
                              sems[p]).wait()

    def combine(s_i, p):
        def tb_body(tb, c2):
            t16 = tb * 16
            if s_i == 0:
                accs = tuple(jnp.zeros((16,), jnp.float32) for _ in range(16))
            else:
                accs = tuple(outb[t16 + l] for l in range(16))

            def j_body(j, acc_t):
                wv = wtb2[p, j, pl.ds(t16, 16)]
                rbase = j * _CHUNK + t16
                return tuple(acc_t[l] + rows2[p, rbase + l] * wv[l]
                             for l in range(16))

            accs = lax.fori_loop(0, 32, j_body, accs)
            for l in range(16):
                outb[t16 + l] = accs[l]
            return c2

        lax.fori_loop(0, _CHUNK // 16, tb_body, 0)

    fire(0, 0, 0)

    def pair_body(c2, carry):
        cc = c2 * 2
        for k in range(6):
            cc_k = cc + k // 3
            s_k = k % 3
            p_k = k % 2
            kn = k + 1
            fire(cc + kn // 3, kn % 3, kn % 2)
            wait_rows(p_k)
            combine(s_k, p_k)
            if s_k == 2:
                pltpu.sync_copy(
                    outb, out_hbm.at[pl.ds(base0 + cc_k * _CHUNK, _CHUNK)])
        return carry

    lax.fori_loop(0, _NCHUNK // 2, pair_body, 0)
    wait_rows(0)


@functools.cache
def _make_sc_gather():
    return pl.kernel(
        _sc_body,
        out_type=jax.ShapeDtypeStruct((_Q, _HD), jnp.float32),
        mesh=plsc.VectorSubcoreMesh(core_axis_name="c", subcore_axis_name="s"),
        scratch_types=[
            pltpu.VMEM((2, 32 * _CHUNK), jnp.int32),
            pltpu.VMEM((2, 32, _CHUNK), jnp.float32),
            pltpu.VMEM((2, 32 * _CHUNK, _HD), jnp.float32),
            pltpu.VMEM((_CHUNK, _HD), jnp.float32),
            pltpu.SemaphoreType.DMA,
            pltpu.SemaphoreType.DMA,
        ],
        compiler_params=pltpu.CompilerParams(use_tc_tiling_on_sc=False),
    )


def _out_kernel(samp_ref, wout_ref, gamma_ref, beta_ref, o_ref):
    sb = samp_ref[...].reshape(_HEADS, _V, _HD)
    acc = jnp.zeros((_CQ, _V), jnp.float32)
    for h in range(_HEADS):
        acc = acc + lax.dot_general(
            wout_ref[:, h * _HD:(h + 1) * _HD], sb[h],
            (((1,), (1,)), ((), ())), preferred_element_type=jnp.float32)
    mean = jnp.mean(acc, axis=1, keepdims=True)
    ctr = acc - mean
    var = jnp.mean(ctr * ctr, axis=1, keepdims=True)
    o_ref[0] = ctr / jnp.sqrt(var + _EPS) * gamma_ref[...] + beta_ref[...]


_out_call = pl.pallas_call(
    _out_kernel,
    grid=(_B,),
    in_specs=[
        pl.BlockSpec((_HEADS * _V, _HD), lambda b: (b, 0)),
        pl.BlockSpec((_CQ, _HID), lambda b: (0, 0)),
        pl.BlockSpec((_CQ, 1), lambda b: (0, 0)),
        pl.BlockSpec((_CQ, 1), lambda b: (0, 0)),
    ],
    out_specs=pl.BlockSpec((1, _CQ, _V), lambda b: (b, 0, 0)),
    out_shape=jax.ShapeDtypeStruct((_B, _CQ, _V), jnp.float32),
    compiler_params=pltpu.CompilerParams(vmem_limit_bytes=110 * 1024 * 1024),
)


def kernel(query_feature, value_0, value_1, value_2, Wq, Wv0, Wv1, Wv2,
           Woff, boff, Ww, bw, Wout, gamma, beta):
    qf = query_feature.reshape(_B, _CQ, _V)
    idxs = []
    wts = []
    for s_i in range(_SCALES):
        i_s, w_s = _make_prep(s_i)(qf, Wq, Woff, boff, Ww, bw)
        idxs.append(i_s)
        wts.append(w_s)

    values = (value_0, value_1, value_2)
    wvs = (Wv0, Wv1, Wv2)
    tabs = []
    for s_i in range(_SCALES):
        svox = _SCALE_DIMS[s_i] ** 3
        nblk = min(svox, 8192)
        vflat = values[s_i].reshape(_B, _SCALE_CH[s_i], svox)
        proj = _make_proj(s_i, nblk)(vflat, wvs[s_i])
        tabs.append(proj.reshape(_B * svox * _HEADS, _HD))

    sampled = _make_sc_gather()(idxs[0], wts[0], idxs[1], wts[1],
                                idxs[2], wts[2], tabs[0], tabs[1], tabs[2])

    out = _out_call(sampled, Wout,
                    gamma.reshape(_CQ, 1), beta.reshape(_CQ, 1))
    return out.reshape(_B, _CQ, _DIM, _DIM, _DIM)
